# baseline (speedup 1.0000x reference)
import dataclasses

import jax
import jax.numpy as jnp
from jax import lax
from jax.experimental import pallas as pl
from jax.experimental.pallas import tpu as pltpu
from jax.experimental.pallas import tpu_sc as plsc

N = 4194304
NC, NS, L = 2, 16, 16
NW = NC * NS
P = N // NW
B = 2048
CH = P // B
STEPS = B // L

_mesh = plsc.VectorSubcoreMesh(core_axis_name="c", subcore_axis_name="s")

_cp = pltpu.CompilerParams()
if "needs_layout_passes" in pltpu.CompilerParams.__dataclass_fields__:
    _cp = dataclasses.replace(_cp, needs_layout_passes=False)


def _body(xs_hbm, data_hbm, out_hbm,
          xyz_v, idx_v, gat_v, wx_v, wy_v, wz_v, out_v, sem):
    wid = lax.axis_index("s") * NC + lax.axis_index("c")
    iota = lax.iota(jnp.int32, L)

    @pl.loop(0, CH)
    def _chunk(g):
        base = wid * P + g * B
        pltpu.sync_copy(xs_hbm.at[pl.ds(base * 3, B * 3)], xyz_v)

        @pl.loop(0, STEPS)
        def _idx(s):
            o = s * L
            r3 = (iota + o) * 3
            xf = plsc.load_gather(xyz_v, [r3]) * 255.0
            yf = plsc.load_gather(xyz_v, [r3 + 1]) * 255.0
            zf = plsc.load_gather(xyz_v, [r3 + 2]) * 255.0
            ix = xf.astype(jnp.int32)
            iy = yf.astype(jnp.int32)
            iz = zf.astype(jnp.int32)
            wx_v[pl.ds(o, L)] = xf - ix.astype(jnp.float32)
            wy_v[pl.ds(o, L)] = yf - iy.astype(jnp.float32)
            wz_v[pl.ds(o, L)] = zf - iz.astype(jnp.float32)
            f000 = (ix << 16) + (iy << 8) + iz
            idx_v[pl.ds(0 * B + o, L)] = f000
            idx_v[pl.ds(1 * B + o, L)] = f000 + 1
            idx_v[pl.ds(2 * B + o, L)] = f000 + 256
            idx_v[pl.ds(3 * B + o, L)] = f000 + 257
            idx_v[pl.ds(4 * B + o, L)] = f000 + 65536
            idx_v[pl.ds(5 * B + o, L)] = f000 + 65537
            idx_v[pl.ds(6 * B + o, L)] = f000 + 65792
            idx_v[pl.ds(7 * B + o, L)] = f000 + 65793

        pltpu.async_copy(data_hbm.at[idx_v], gat_v, sem).wait()

        @pl.loop(0, STEPS)
        def _comb(s):
            o = s * L
            wx = wx_v[pl.ds(o, L)]
            wy = wy_v[pl.ds(o, L)]
            wz = wz_v[pl.ds(o, L)]
            c000 = gat_v[pl.ds(0 * B + o, L)]
            c001 = gat_v[pl.ds(1 * B + o, L)]
            c010 = gat_v[pl.ds(2 * B + o, L)]
            c011 = gat_v[pl.ds(3 * B + o, L)]
            c100 = gat_v[pl.ds(4 * B + o, L)]
            c101 = gat_v[pl.ds(5 * B + o, L)]
            c110 = gat_v[pl.ds(6 * B + o, L)]
            c111 = gat_v[pl.ds(7 * B + o, L)]
            c00 = c000 + wz * (c001 - c000)
            c01 = c010 + wz * (c011 - c010)
            c10 = c100 + wz * (c101 - c100)
            c11 = c110 + wz * (c111 - c110)
            c0 = c00 + wy * (c01 - c00)
            c1 = c10 + wy * (c11 - c10)
            out_v[pl.ds(o, L)] = c0 + wx * (c1 - c0)

        pltpu.sync_copy(out_v, out_hbm.at[pl.ds(base, B)])


def kernel(xs, data):
    xs_flat = xs.reshape(-1)
    data_flat = data.reshape(-1)
    run = pl.kernel(
        _body,
        out_type=jax.ShapeDtypeStruct((N,), jnp.float32),
        mesh=_mesh,
        scratch_types=[
            pltpu.VMEM((B * 3,), jnp.float32),
            pltpu.VMEM((8 * B,), jnp.int32),
            pltpu.VMEM((8 * B,), jnp.float32),
            pltpu.VMEM((B,), jnp.float32),
            pltpu.VMEM((B,), jnp.float32),
            pltpu.VMEM((B,), jnp.float32),
            pltpu.VMEM((B,), jnp.float32),
            pltpu.SemaphoreType.DMA,
        ],
        compiler_params=_cp,
    )
    return run(xs_flat, data_flat)


# speedup vs baseline: 1.0001x; 1.0001x over previous
import dataclasses

import jax
import jax.numpy as jnp
from jax import lax
from jax.experimental import pallas as pl
from jax.experimental.pallas import tpu as pltpu
from jax.experimental.pallas import tpu_sc as plsc

N = 4194304
NC, NS, L = 2, 16, 16
NW = NC * NS
P = N // NW
B = 2048
CH = P // B
STEPS = B // L

_mesh = plsc.VectorSubcoreMesh(core_axis_name="c", subcore_axis_name="s")

_cp = pltpu.CompilerParams()
if "needs_layout_passes" in pltpu.CompilerParams.__dataclass_fields__:
    _cp = dataclasses.replace(_cp, needs_layout_passes=False)


def _body(xs_hbm, data_hbm, out_hbm,
          xyz_v, idx_v, gat_v, wx_v, wy_v, wz_v, out_v, sem):
    wid = lax.axis_index("s") * NC + lax.axis_index("c")
    iota = lax.iota(jnp.int32, L)

    @pl.loop(0, CH)
    def _chunk(g):
        base = wid * P + g * B
        pltpu.sync_copy(xs_hbm.at[pl.ds(base * 3, B * 3)], xyz_v)

        @pl.loop(0, STEPS)
        def _idx(s):
            o = s * L
            r3 = (iota + o) * 3
            xf = plsc.load_gather(xyz_v, [r3]) * 255.0
            yf = plsc.load_gather(xyz_v, [r3 + 1]) * 255.0
            zf = plsc.load_gather(xyz_v, [r3 + 2]) * 255.0
            ix = xf.astype(jnp.int32)
            iy = yf.astype(jnp.int32)
            iz = zf.astype(jnp.int32)
            wx_v[pl.ds(o, L)] = xf - ix.astype(jnp.float32)
            wy_v[pl.ds(o, L)] = yf - iy.astype(jnp.float32)
            wz_v[pl.ds(o, L)] = zf - iz.astype(jnp.float32)
            f000 = (ix << 16) + (iy << 8) + iz
            idx_v[pl.ds(0 * B + o, L)] = f000
            idx_v[pl.ds(1 * B + o, L)] = f000 + 1
            idx_v[pl.ds(2 * B + o, L)] = f000 + 256
            idx_v[pl.ds(3 * B + o, L)] = f000 + 257
            idx_v[pl.ds(4 * B + o, L)] = f000 + 65536
            idx_v[pl.ds(5 * B + o, L)] = f000 + 65537
            idx_v[pl.ds(6 * B + o, L)] = f000 + 65792
            idx_v[pl.ds(7 * B + o, L)] = f000 + 65793

        copies = [
            pltpu.async_copy(
                data_hbm.at[idx_v.at[pl.ds(c * B, B)]],
                gat_v.at[pl.ds(c * B, B)], sem)
            for c in range(8)
        ]
        for cp in copies:
            cp.wait()

        @pl.loop(0, STEPS)
        def _comb(s):
            o = s * L
            wx = wx_v[pl.ds(o, L)]
            wy = wy_v[pl.ds(o, L)]
            wz = wz_v[pl.ds(o, L)]
            c000 = gat_v[pl.ds(0 * B + o, L)]
            c001 = gat_v[pl.ds(1 * B + o, L)]
            c010 = gat_v[pl.ds(2 * B + o, L)]
            c011 = gat_v[pl.ds(3 * B + o, L)]
            c100 = gat_v[pl.ds(4 * B + o, L)]
            c101 = gat_v[pl.ds(5 * B + o, L)]
            c110 = gat_v[pl.ds(6 * B + o, L)]
            c111 = gat_v[pl.ds(7 * B + o, L)]
            c00 = c000 + wz * (c001 - c000)
            c01 = c010 + wz * (c011 - c010)
            c10 = c100 + wz * (c101 - c100)
            c11 = c110 + wz * (c111 - c110)
            c0 = c00 + wy * (c01 - c00)
            c1 = c10 + wy * (c11 - c10)
            out_v[pl.ds(o, L)] = c0 + wx * (c1 - c0)

        pltpu.sync_copy(out_v, out_hbm.at[pl.ds(base, B)])


def kernel(xs, data):
    xs_flat = xs.reshape(-1)
    data_flat = data.reshape(-1)
    run = pl.kernel(
        _body,
        out_type=jax.ShapeDtypeStruct((N,), jnp.float32),
        mesh=_mesh,
        scratch_types=[
            pltpu.VMEM((B * 3,), jnp.float32),
            pltpu.VMEM((8 * B,), jnp.int32),
            pltpu.VMEM((8 * B,), jnp.float32),
            pltpu.VMEM((B,), jnp.float32),
            pltpu.VMEM((B,), jnp.float32),
            pltpu.VMEM((B,), jnp.float32),
            pltpu.VMEM((B,), jnp.float32),
            pltpu.SemaphoreType.DMA,
        ],
        compiler_params=_cp,
    )
    return run(xs_flat, data_flat)


# speedup vs baseline: 1.0251x; 1.0250x over previous
import dataclasses

import jax
import jax.numpy as jnp
from jax import lax
from jax.experimental import pallas as pl
from jax.experimental.pallas import tpu as pltpu
from jax.experimental.pallas import tpu_sc as plsc

N = 4194304
NC, NS, L = 2, 16, 16
NW = NC * NS
P = N // NW
B = 2048
CH = P // B
STEPS = B // L

_mesh = plsc.VectorSubcoreMesh(core_axis_name="c", subcore_axis_name="s")

_cp = pltpu.CompilerParams()
if "needs_layout_passes" in pltpu.CompilerParams.__dataclass_fields__:
    _cp = dataclasses.replace(_cp, needs_layout_passes=False)


def _body(xs_hbm, data_hbm, out_hbm,
          xyz_v, idx_v, gat_v, wx_v, wy_v, wz_v, out_v, sem):
    wid = lax.axis_index("s") * NC + lax.axis_index("c")
    iota = lax.iota(jnp.int32, L)

    @pl.loop(0, CH)
    def _chunk(g):
        base = wid * P + g * B
        pltpu.sync_copy(xs_hbm.at[pl.ds(base * 3, B * 3)], xyz_v)

        @plsc.parallel_loop(0, STEPS, unroll=4)
        def _idx(s):
            o = s * L
            r3 = (iota + o) * 3
            xf = plsc.load_gather(xyz_v, [r3]) * 255.0
            yf = plsc.load_gather(xyz_v, [r3 + 1]) * 255.0
            zf = plsc.load_gather(xyz_v, [r3 + 2]) * 255.0
            ix = xf.astype(jnp.int32)
            iy = yf.astype(jnp.int32)
            iz = zf.astype(jnp.int32)
            wx_v[pl.ds(o, L)] = xf - ix.astype(jnp.float32)
            wy_v[pl.ds(o, L)] = yf - iy.astype(jnp.float32)
            wz_v[pl.ds(o, L)] = zf - iz.astype(jnp.float32)
            f000 = (ix << 16) + (iy << 8) + iz
            idx_v[pl.ds(0 * B + o, L)] = f000
            idx_v[pl.ds(1 * B + o, L)] = f000 + 1
            idx_v[pl.ds(2 * B + o, L)] = f000 + 256
            idx_v[pl.ds(3 * B + o, L)] = f000 + 257
            idx_v[pl.ds(4 * B + o, L)] = f000 + 65536
            idx_v[pl.ds(5 * B + o, L)] = f000 + 65537
            idx_v[pl.ds(6 * B + o, L)] = f000 + 65792
            idx_v[pl.ds(7 * B + o, L)] = f000 + 65793

        pltpu.async_copy(data_hbm.at[idx_v], gat_v, sem).wait()

        @plsc.parallel_loop(0, STEPS, unroll=4)
        def _comb(s):
            o = s * L
            wx = wx_v[pl.ds(o, L)]
            wy = wy_v[pl.ds(o, L)]
            wz = wz_v[pl.ds(o, L)]
            c000 = gat_v[pl.ds(0 * B + o, L)]
            c001 = gat_v[pl.ds(1 * B + o, L)]
            c010 = gat_v[pl.ds(2 * B + o, L)]
            c011 = gat_v[pl.ds(3 * B + o, L)]
            c100 = gat_v[pl.ds(4 * B + o, L)]
            c101 = gat_v[pl.ds(5 * B + o, L)]
            c110 = gat_v[pl.ds(6 * B + o, L)]
            c111 = gat_v[pl.ds(7 * B + o, L)]
            c00 = c000 + wz * (c001 - c000)
            c01 = c010 + wz * (c011 - c010)
            c10 = c100 + wz * (c101 - c100)
            c11 = c110 + wz * (c111 - c110)
            c0 = c00 + wy * (c01 - c00)
            c1 = c10 + wy * (c11 - c10)
            out_v[pl.ds(o, L)] = c0 + wx * (c1 - c0)

        pltpu.sync_copy(out_v, out_hbm.at[pl.ds(base, B)])


def kernel(xs, data):
    xs_flat = xs.reshape(-1)
    data_flat = data.reshape(-1)
    run = pl.kernel(
        _body,
        out_type=jax.ShapeDtypeStruct((N,), jnp.float32),
        mesh=_mesh,
        scratch_types=[
            pltpu.VMEM((B * 3,), jnp.float32),
            pltpu.VMEM((8 * B,), jnp.int32),
            pltpu.VMEM((8 * B,), jnp.float32),
            pltpu.VMEM((B,), jnp.float32),
            pltpu.VMEM((B,), jnp.float32),
            pltpu.VMEM((B,), jnp.float32),
            pltpu.VMEM((B,), jnp.float32),
            pltpu.SemaphoreType.DMA,
        ],
        compiler_params=_cp,
    )
    return run(xs_flat, data_flat)


# speedup vs baseline: 2.8111x; 2.7423x over previous
import dataclasses

import jax
import jax.numpy as jnp
from jax import lax
from jax.experimental import pallas as pl
from jax.experimental.pallas import tpu as pltpu
from jax.experimental.pallas import tpu_sc as plsc

N = 4194304
NC, NS, L = 2, 16, 16
NW = NC * NS
P = N // NW
B = 1024
CH = P // B
STEPS = B // L

_mesh = plsc.VectorSubcoreMesh(core_axis_name="c", subcore_axis_name="s")

_cp = pltpu.CompilerParams()
if "needs_layout_passes" in pltpu.CompilerParams.__dataclass_fields__:
    _cp = dataclasses.replace(_cp, needs_layout_passes=False)


def _body(xs_hbm, data_hbm, out_hbm,
          xv, yv, zv, idx_v, gat_v, wx_v, wy_v, wz_v, out_v, sem):
    wid = lax.axis_index("s") * NC + lax.axis_index("c")

    @pl.loop(0, CH)
    def _chunk(g):
        base = wid * P + g * B
        pltpu.sync_copy(xs_hbm.at[pl.ds(0 * N + base, B)], xv)
        pltpu.sync_copy(xs_hbm.at[pl.ds(1 * N + base, B)], yv)
        pltpu.sync_copy(xs_hbm.at[pl.ds(2 * N + base, B)], zv)

        for s in range(STEPS):
            o = s * L
            xf = xv[pl.ds(o, L)] * 255.0
            yf = yv[pl.ds(o, L)] * 255.0
            zf = zv[pl.ds(o, L)] * 255.0
            ix = xf.astype(jnp.int32)
            iy = yf.astype(jnp.int32)
            iz = zf.astype(jnp.int32)
            wx_v[pl.ds(o, L)] = xf - ix.astype(jnp.float32)
            wy_v[pl.ds(o, L)] = yf - iy.astype(jnp.float32)
            wz_v[pl.ds(o, L)] = zf - iz.astype(jnp.float32)
            f000 = (ix << 16) + (iy << 8) + iz
            idx_v[pl.ds(0 * B + o, L)] = f000
            idx_v[pl.ds(1 * B + o, L)] = f000 + 1
            idx_v[pl.ds(2 * B + o, L)] = f000 + 256
            idx_v[pl.ds(3 * B + o, L)] = f000 + 257
            idx_v[pl.ds(4 * B + o, L)] = f000 + 65536
            idx_v[pl.ds(5 * B + o, L)] = f000 + 65537
            idx_v[pl.ds(6 * B + o, L)] = f000 + 65792
            idx_v[pl.ds(7 * B + o, L)] = f000 + 65793

        pltpu.async_copy(data_hbm.at[idx_v], gat_v, sem).wait()

        for s in range(STEPS):
            o = s * L
            wx = wx_v[pl.ds(o, L)]
            wy = wy_v[pl.ds(o, L)]
            wz = wz_v[pl.ds(o, L)]
            c000 = gat_v[pl.ds(0 * B + o, L)]
            c001 = gat_v[pl.ds(1 * B + o, L)]
            c010 = gat_v[pl.ds(2 * B + o, L)]
            c011 = gat_v[pl.ds(3 * B + o, L)]
            c100 = gat_v[pl.ds(4 * B + o, L)]
            c101 = gat_v[pl.ds(5 * B + o, L)]
            c110 = gat_v[pl.ds(6 * B + o, L)]
            c111 = gat_v[pl.ds(7 * B + o, L)]
            c00 = c000 + wz * (c001 - c000)
            c01 = c010 + wz * (c011 - c010)
            c10 = c100 + wz * (c101 - c100)
            c11 = c110 + wz * (c111 - c110)
            c0 = c00 + wy * (c01 - c00)
            c1 = c10 + wy * (c11 - c10)
            out_v[pl.ds(o, L)] = c0 + wx * (c1 - c0)

        pltpu.sync_copy(out_v, out_hbm.at[pl.ds(base, B)])


def kernel(xs, data):
    xs_t = xs.T.reshape(-1)
    data_flat = data.reshape(-1)
    run = pl.kernel(
        _body,
        out_type=jax.ShapeDtypeStruct((N,), jnp.float32),
        mesh=_mesh,
        scratch_types=[
            pltpu.VMEM((B,), jnp.float32),
            pltpu.VMEM((B,), jnp.float32),
            pltpu.VMEM((B,), jnp.float32),
            pltpu.VMEM((8 * B,), jnp.int32),
            pltpu.VMEM((8 * B,), jnp.float32),
            pltpu.VMEM((B,), jnp.float32),
            pltpu.VMEM((B,), jnp.float32),
            pltpu.VMEM((B,), jnp.float32),
            pltpu.VMEM((B,), jnp.float32),
            pltpu.SemaphoreType.DMA,
        ],
        compiler_params=_cp,
    )
    return run(xs_t, data_flat)
